# baseline (speedup 1.0000x reference)
import functools

import jax
import jax.numpy as jnp
from jax.experimental import pallas as pl
from jax.experimental.pallas import tpu as pltpu

B = 512
T = 32
NC = 50
DN = 20
FD = 768
FD2 = 768
HID = 512
CLS = 512
E = 8
MIN = CLS * 3
MH = 1024
MO = CLS

BB = 32
MB = 128
F32 = jnp.float32


def _dot(a, b, precision=jax.lax.Precision.DEFAULT):
    return jax.lax.dot_general(a, b, (((1,), (0,)), ((), ())),
                               preferred_element_type=F32,
                               precision=precision)


def _enc_body(v_ref, c_ref, asr_ref, top_ref, ttl_ref, aut_ref,
              vidW, vidb, asrW, asrb, topW, topb, ttlW, ttlb, autW, autb,
              comW, comb, f1_ref, f2_ref, tpf_ref):
    relu = jax.nn.relu
    vf = relu(_dot(v_ref[...], vidW[...]) + vidb[...])
    r = jax.lax.broadcasted_iota(jnp.int32, (BB, BB * T), 0)
    c = jax.lax.broadcasted_iota(jnp.int32, (BB, BB * T), 1)
    Sv = jnp.where(c // T == r, 1.0 / T, 0.0).astype(F32)
    vfm = _dot(Sv, vf, jax.lax.Precision.HIGHEST)

    cf = relu(_dot(c_ref[...], comW[...]) + comb[...])
    r2 = jax.lax.broadcasted_iota(jnp.int32, (BB, BB * DN), 0)
    c2 = jax.lax.broadcasted_iota(jnp.int32, (BB, BB * DN), 1)
    Sc = jnp.where(c2 // DN == r2, 1.0 / DN, 0.0).astype(F32)
    cfm = _dot(Sc, cf, jax.lax.Precision.HIGHEST)

    af = relu(_dot(asr_ref[...], asrW[...]) + asrb[...])
    tpf = relu(_dot(top_ref[...], topW[...]) + topb[...])
    tf = relu(_dot(ttl_ref[...], ttlW[...]) + ttlb[...])
    auf = relu(_dot(aut_ref[...], autW[...]) + autb[...])

    f1_ref[:, 0:HID] = vfm
    f1_ref[:, HID:2 * HID] = af
    f1_ref[:, 2 * HID:3 * HID] = tf
    f1_ref[:, 3 * HID:4 * HID] = auf
    f2_ref[:, 0:HID] = vfm
    f2_ref[:, HID:2 * HID] = af
    f2_ref[:, 2 * HID:4 * HID] = cfm
    tpf_ref[...] = tpf


def _encoders(video2, com2, asr, top, ttl, aut, p):
    nblk = B // BB
    row = lambda i: (i, 0)
    full = lambda i: (0, 0)
    out = pl.pallas_call(
        _enc_body,
        grid=(nblk,),
        in_specs=[
            pl.BlockSpec((BB * T, FD), row),
            pl.BlockSpec((BB * DN, FD), row),
            pl.BlockSpec((BB, FD), row),
            pl.BlockSpec((BB, FD2), row),
            pl.BlockSpec((BB, FD), row),
            pl.BlockSpec((BB, FD), row),
            pl.BlockSpec((FD, HID), full), pl.BlockSpec((1, HID), full),
            pl.BlockSpec((FD, HID), full), pl.BlockSpec((1, HID), full),
            pl.BlockSpec((FD2, HID), full), pl.BlockSpec((1, HID), full),
            pl.BlockSpec((FD, HID), full), pl.BlockSpec((1, HID), full),
            pl.BlockSpec((FD, HID), full), pl.BlockSpec((1, HID), full),
            pl.BlockSpec((FD, 2 * HID), full), pl.BlockSpec((1, 2 * HID), full),
        ],
        out_specs=[
            pl.BlockSpec((BB, 4 * HID), row),
            pl.BlockSpec((BB, 4 * HID), row),
            pl.BlockSpec((BB, HID), row),
        ],
        out_shape=[
            jax.ShapeDtypeStruct((B, 4 * HID), F32),
            jax.ShapeDtypeStruct((B, 4 * HID), F32),
            jax.ShapeDtypeStruct((B, HID), F32),
        ],
    )(video2, com2, asr, top, ttl, aut,
      p["vid_W"], p["vid_b"].reshape(1, -1),
      p["asr_W"], p["asr_b"].reshape(1, -1),
      p["top_W"], p["top_b"].reshape(1, -1),
      p["ttl_W"], p["ttl_b"].reshape(1, -1),
      p["aut_W"], p["aut_b"].reshape(1, -1),
      p["com_W"], p["com_b"].reshape(1, -1))
    return out


def _mlp_ln_body(x_ref, W1, b1, g, be, W2, b2, out_ref):
    t = _dot(x_ref[...], W1[...]) + b1[...]
    m = t.mean(-1, keepdims=True)
    v = ((t - m) ** 2).mean(-1, keepdims=True)
    t = (t - m) / jnp.sqrt(v + 1e-5) * g[...] + be[...]
    t = jax.nn.relu(t)
    out_ref[...] = _dot(t, W2[...]) + b2[...]


def _mlp_ln(x, W1, b1, g, be, W2, b2):
    din, dh = W1.shape
    dout = W2.shape[1]
    row = lambda i: (i, 0)
    full = lambda i: (0, 0)
    return pl.pallas_call(
        _mlp_ln_body,
        grid=(B // MB,),
        in_specs=[pl.BlockSpec((MB, din), row),
                  pl.BlockSpec((din, dh), full),
                  pl.BlockSpec((1, dh), full),
                  pl.BlockSpec((1, dh), full),
                  pl.BlockSpec((1, dh), full),
                  pl.BlockSpec((dh, dout), full),
                  pl.BlockSpec((1, dout), full)],
        out_specs=pl.BlockSpec((MB, dout), row),
        out_shape=jax.ShapeDtypeStruct((B, dout), F32),
    )(x, W1, b1.reshape(1, -1), g.reshape(1, -1), be.reshape(1, -1),
      W2, b2.reshape(1, -1))


def _router_stats(gates):
    importance = gates.sum(0, keepdims=True)
    load = (gates > 0.0).astype(F32).sum(0, keepdims=True)

    def cv_sq(v):
        m = v.mean()
        var = ((v - m) ** 2).mean()
        return var / (m * m + 1e-10)

    return (cv_sq(importance) + cv_sq(load)) * 0.01


def _top2_gates(logits):
    lane = jax.lax.broadcasted_iota(jnp.int32, logits.shape, 1)
    i1 = jnp.argmax(logits, axis=1, keepdims=True)
    oh1 = (lane == i1)
    m1 = jnp.max(logits, axis=1, keepdims=True)
    masked = jnp.where(oh1, -jnp.inf, logits)
    i2 = jnp.argmax(masked, axis=1, keepdims=True)
    oh2 = (lane == i2)
    m2 = jnp.max(masked, axis=1, keepdims=True)
    e2 = jnp.exp(m2 - m1)
    den = 1.0 + e2
    g1 = 1.0 / den
    g2 = e2 / den
    return jnp.where(oh1, g1, 0.0) + jnp.where(oh2, g2, 0.0)


def _reduce_body(x_ref, rdW, rdb, fea_ref):
    fea_ref[...] = _dot(x_ref[...], rdW[...]) + rdb[...]


def _reduce(x7, p):
    row = lambda i: (i, 0)
    full = lambda i: (0, 0)
    return pl.pallas_call(
        _reduce_body,
        grid=(B // MB,),
        in_specs=[pl.BlockSpec((MB, 7 * CLS), row),
                  pl.BlockSpec((7 * CLS, MIN), full),
                  pl.BlockSpec((1, MIN), full)],
        out_specs=pl.BlockSpec((MB, MIN), row),
        out_shape=jax.ShapeDtypeStruct((B, MIN), F32),
    )(x7, p["rd_W"], p["rd_b"].reshape(1, -1))


def _router_body(fea_ref, wgc, wgr, gc_ref, gr_ref, lc_ref, lr_ref):
    fea = fea_ref[...]
    gc = _top2_gates(_dot(fea, wgc[...]))
    gr = _top2_gates(_dot(fea, wgr[...]))
    gc_ref[...] = gc
    gr_ref[...] = gr
    lc_ref[0, 0] = _router_stats(gc)
    lr_ref[0, 0] = _router_stats(gr)


def _router(fea, p):
    full = lambda: (0, 0)
    return pl.pallas_call(
        _router_body,
        in_specs=[pl.BlockSpec((B, MIN), full),
                  pl.BlockSpec((MIN, E), full),
                  pl.BlockSpec((MIN, E), full)],
        out_specs=[pl.BlockSpec((B, E), full),
                   pl.BlockSpec((B, E), full),
                   pl.BlockSpec(memory_space=pltpu.SMEM),
                   pl.BlockSpec(memory_space=pltpu.SMEM)],
        out_shape=[jax.ShapeDtypeStruct((B, E), F32),
                   jax.ShapeDtypeStruct((B, E), F32),
                   jax.ShapeDtypeStruct((1, 1), F32),
                   jax.ShapeDtypeStruct((1, 1), F32)],
    )(fea, p["wg_cls"], p["wg_reg"])


def _moe_body(fea_ref, g_ref, W1, b1, W2, b2, out_ref):
    e = pl.program_id(0)
    x = fea_ref[...]
    lane = jax.lax.broadcasted_iota(jnp.int32, (B, E), 1)
    sel = (lane == e).astype(F32)

    h = jax.nn.relu(_dot(x, W1[0]) + b1[0])
    y = _dot(h, W2[0]) + b2[0]
    col = (g_ref[...] * sel).sum(1, keepdims=True)
    y = y.astype(jnp.bfloat16).astype(F32)
    col = col.astype(jnp.bfloat16).astype(F32)

    @pl.when(e == 0)
    def _():
        out_ref[...] = col * y

    @pl.when(e != 0)
    def _():
        out_ref[...] += col * y


def _moe_branch_pallas(fea, g, W1, b1, W2, b2):
    full2 = lambda e: (0, 0)
    we3 = lambda e: (e, 0, 0)
    return pl.pallas_call(
        _moe_body,
        grid=(E,),
        in_specs=[pl.BlockSpec((B, MIN), full2),
                  pl.BlockSpec((B, E), full2),
                  pl.BlockSpec((1, MIN, MH), we3), pl.BlockSpec((1, 1, MH), we3),
                  pl.BlockSpec((1, MH, MO), we3), pl.BlockSpec((1, 1, MO), we3)],
        out_specs=pl.BlockSpec((B, MO), full2),
        out_shape=jax.ShapeDtypeStruct((B, MO), F32),
    )(fea, g, W1, b1.reshape(E, 1, MH), W2, b2.reshape(E, 1, MO))


def _head_body(x_ref, W1, b1, g, be, W2, b2, out_ref):
    t = _dot(x_ref[...], W1[...]) + b1[...]
    m = t.mean(-1, keepdims=True)
    v = ((t - m) ** 2).mean(-1, keepdims=True)
    t = jax.nn.relu((t - m) / jnp.sqrt(v + 1e-5) * g[...] + be[...])
    out_ref[...] = _dot(t, W2[...]) + b2[...]


def _head(x, W1, b1, g, be, W2, b2):
    din, dh = W1.shape
    dout = W2.shape[1]
    return pl.pallas_call(
        _head_body,
        in_specs=[pl.BlockSpec((B, din), lambda: (0, 0)),
                  pl.BlockSpec((din, dh), lambda: (0, 0)),
                  pl.BlockSpec((1, dh), lambda: (0, 0)),
                  pl.BlockSpec((1, dh), lambda: (0, 0)),
                  pl.BlockSpec((1, dh), lambda: (0, 0)),
                  pl.BlockSpec((dh, dout), lambda: (0, 0)),
                  pl.BlockSpec((1, dout), lambda: (0, 0))],
        out_specs=pl.BlockSpec((B, dout), lambda: (0, 0)),
        out_shape=jax.ShapeDtypeStruct((B, dout), F32),
    )(x, W1, b1.reshape(1, -1), g.reshape(1, -1), be.reshape(1, -1),
      W2, b2.reshape(1, -1))


def _dot_nt(a, b):
    return jax.lax.dot_general(a, b, (((1,), (1,)), ((), ())),
                               preferred_element_type=F32,
                               precision=jax.lax.Precision.DEFAULT)


def _gru_body(x_ref, wih, whh, bih, bhh, fcW, fcb,
              rgW1, rgb1, rgg, rgbe, rgW2, rgb2, out_ref, gout_ref, X, GI):
    X[...] = x_ref[...]
    for l in range(3):
        GI[...] = _dot_nt(X[...], wih[l]) + bih[l][None, :]
        bhh_l = bhh[l][None, :]
        whh_l = whh[l]

        def step(t, h):
            gh = _dot_nt(h, whh_l) + bhh_l
            gi = GI[pl.ds(t, 1), :]
            r = jax.nn.sigmoid(gi[:, :CLS] + gh[:, :CLS])
            z = jax.nn.sigmoid(gi[:, CLS:2 * CLS] + gh[:, CLS:2 * CLS])
            n = jnp.tanh(gi[:, 2 * CLS:] + r * gh[:, 2 * CLS:])
            hnew = (1.0 - z) * n + z * h
            X[pl.ds(t, 1), :] = hnew
            return hnew

        jax.lax.fori_loop(0, B, step, jnp.zeros((1, CLS), F32))

    g = _dot(X[...], fcW[...]) + fcb[...]
    gout_ref[...] = g
    t = _dot(g, rgW1[...]) + rgb1[...]
    m = t.mean(-1, keepdims=True)
    v = ((t - m) ** 2).mean(-1, keepdims=True)
    t = jax.nn.relu((t - m) / jnp.sqrt(v + 1e-5) * rgg[...] + rgbe[...])
    out_ref[...] = _dot(t, rgW2[...]) + rgb2[...]


def _gru_reg(reg_fea, p, rgW2p, rgb2p):
    f = lambda: (0, 0)
    f3 = lambda: (0, 0, 0)
    out, _ = pl.pallas_call(
        _gru_body,
        in_specs=[pl.BlockSpec((B, CLS), f),
                  pl.BlockSpec((3, 3 * CLS, CLS), f3),
                  pl.BlockSpec((3, 3 * CLS, CLS), f3),
                  pl.BlockSpec((3, 3 * CLS), f),
                  pl.BlockSpec((3, 3 * CLS), f),
                  pl.BlockSpec((CLS, CLS), f),
                  pl.BlockSpec((1, CLS), f),
                  pl.BlockSpec((CLS, CLS // 2), f),
                  pl.BlockSpec((1, CLS // 2), f),
                  pl.BlockSpec((1, CLS // 2), f),
                  pl.BlockSpec((1, CLS // 2), f),
                  pl.BlockSpec((CLS // 2, 128), f),
                  pl.BlockSpec((1, 128), f)],
        out_specs=[pl.BlockSpec((B, 128), f),
                   pl.BlockSpec((B, CLS), f)],
        out_shape=[jax.ShapeDtypeStruct((B, 128), F32),
                   jax.ShapeDtypeStruct((B, CLS), F32)],
        scratch_shapes=[pltpu.VMEM((B, CLS), F32),
                        pltpu.VMEM((B, 3 * CLS), F32)],
    )(reg_fea, p["gru_Wih"], p["gru_Whh"], p["gru_bih"], p["gru_bhh"],
      p["fc_W"], p["fc_b"].reshape(1, -1),
      p["rg_W1"], p["rg_b1"].reshape(1, -1),
      p["rg_g"].reshape(1, -1), p["rg_be"].reshape(1, -1),
      rgW2p, rgb2p)
    return out


def kernel(video_feas, asr_feas, topics_fea, title_feas, author_feas,
           comment_feas, params):
    p = params
    video2 = video_feas.reshape(B * T, FD)
    com2 = comment_feas[:, :DN, :].reshape(B * DN, FD)

    f1, f2, tpf = _encoders(video2, com2, asr_feas, topics_fea,
                            title_feas, author_feas, p)

    v_p = _mlp_ln(f1, p["vp_W1"], p["vp_b1"], p["vp_g"], p["vp_be"],
                  p["vp_W2"], p["vp_b2"])
    v_c = _mlp_ln(f2, p["vc_W1"], p["vc_b1"], p["vc_g"], p["vc_be"],
                  p["vc_W2"], p["vc_b2"])

    x7 = jnp.concatenate([tpf, v_p, v_c], axis=1)
    fea = _reduce(x7, p)
    gc, gr, lc, lr = _router(fea, p)

    cls_fea = _moe_branch_pallas(fea, gc, p["ec_W1"], p["ec_b1"],
                                 p["ec_W2"], p["ec_b2"])
    reg_fea = _moe_branch_pallas(fea, gr, p["er_W1"], p["er_b1"],
                                 p["er_W2"], p["er_b2"])

    clW2p = jnp.pad(p["cl_W2"], ((0, 0), (0, 126)))
    clb2p = jnp.pad(p["cl_b2"], (0, 126)).reshape(1, -1)
    cls_pad = _head(cls_fea, p["cl_W1"], p["cl_b1"], p["cl_g"], p["cl_be"],
                    clW2p, clb2p.reshape(-1))
    cls_output = cls_pad[:, :2]

    rgW2p = jnp.pad(p["rg_W2"], ((0, 0), (0, 127)))
    rgb2p = jnp.pad(p["rg_b2"], (0, 127)).reshape(1, -1)
    reg_pad = _gru_reg(reg_fea, p, rgW2p, rgb2p)
    reg_output = reg_pad[:, :1]

    return (cls_output, reg_output, lc.reshape(()), lr.reshape(()))


# speedup vs baseline: 1.3530x; 1.3530x over previous
import functools

import jax
import jax.numpy as jnp
from jax.experimental import pallas as pl
from jax.experimental.pallas import tpu as pltpu

B = 512
T = 32
NC = 50
DN = 20
FD = 768
FD2 = 768
HID = 512
CLS = 512
E = 8
MIN = CLS * 3
MH = 1024
MO = CLS

BB = 32
MB = 128
F32 = jnp.float32


def _dot(a, b, precision=jax.lax.Precision.DEFAULT):
    return jax.lax.dot_general(a, b, (((1,), (0,)), ((), ())),
                               preferred_element_type=F32,
                               precision=precision)


def _enc_body(v_ref, c_ref, asr_ref, top_ref, ttl_ref, aut_ref,
              vidW, vidb, asrW, asrb, topW, topb, ttlW, ttlb, autW, autb,
              comW, comb, f1_ref, f2_ref, tpf_ref):
    relu = jax.nn.relu
    vf = relu(_dot(v_ref[...], vidW[...]) + vidb[...])
    r = jax.lax.broadcasted_iota(jnp.int32, (BB, BB * T), 0)
    c = jax.lax.broadcasted_iota(jnp.int32, (BB, BB * T), 1)
    Sv = jnp.where(c // T == r, 1.0 / T, 0.0).astype(F32)
    vfm = _dot(Sv, vf, jax.lax.Precision.HIGHEST)

    cf = relu(_dot(c_ref[...], comW[...]) + comb[...])
    r2 = jax.lax.broadcasted_iota(jnp.int32, (BB, BB * DN), 0)
    c2 = jax.lax.broadcasted_iota(jnp.int32, (BB, BB * DN), 1)
    Sc = jnp.where(c2 // DN == r2, 1.0 / DN, 0.0).astype(F32)
    cfm = _dot(Sc, cf, jax.lax.Precision.HIGHEST)

    af = relu(_dot(asr_ref[...], asrW[...]) + asrb[...])
    tpf = relu(_dot(top_ref[...], topW[...]) + topb[...])
    tf = relu(_dot(ttl_ref[...], ttlW[...]) + ttlb[...])
    auf = relu(_dot(aut_ref[...], autW[...]) + autb[...])

    f1_ref[:, 0:HID] = vfm
    f1_ref[:, HID:2 * HID] = af
    f1_ref[:, 2 * HID:3 * HID] = tf
    f1_ref[:, 3 * HID:4 * HID] = auf
    f2_ref[:, 0:HID] = vfm
    f2_ref[:, HID:2 * HID] = af
    f2_ref[:, 2 * HID:4 * HID] = cfm
    tpf_ref[...] = tpf


def _encoders(video2, com2, asr, top, ttl, aut, p):
    nblk = B // BB
    row = lambda i: (i, 0)
    full = lambda i: (0, 0)
    out = pl.pallas_call(
        _enc_body,
        grid=(nblk,),
        in_specs=[
            pl.BlockSpec((BB * T, FD), row),
            pl.BlockSpec((BB * DN, FD), row),
            pl.BlockSpec((BB, FD), row),
            pl.BlockSpec((BB, FD2), row),
            pl.BlockSpec((BB, FD), row),
            pl.BlockSpec((BB, FD), row),
            pl.BlockSpec((FD, HID), full), pl.BlockSpec((1, HID), full),
            pl.BlockSpec((FD, HID), full), pl.BlockSpec((1, HID), full),
            pl.BlockSpec((FD2, HID), full), pl.BlockSpec((1, HID), full),
            pl.BlockSpec((FD, HID), full), pl.BlockSpec((1, HID), full),
            pl.BlockSpec((FD, HID), full), pl.BlockSpec((1, HID), full),
            pl.BlockSpec((FD, 2 * HID), full), pl.BlockSpec((1, 2 * HID), full),
        ],
        out_specs=[
            pl.BlockSpec((BB, 4 * HID), row),
            pl.BlockSpec((BB, 4 * HID), row),
            pl.BlockSpec((BB, HID), row),
        ],
        out_shape=[
            jax.ShapeDtypeStruct((B, 4 * HID), F32),
            jax.ShapeDtypeStruct((B, 4 * HID), F32),
            jax.ShapeDtypeStruct((B, HID), F32),
        ],
    )(video2, com2, asr, top, ttl, aut,
      p["vid_W"], p["vid_b"].reshape(1, -1),
      p["asr_W"], p["asr_b"].reshape(1, -1),
      p["top_W"], p["top_b"].reshape(1, -1),
      p["ttl_W"], p["ttl_b"].reshape(1, -1),
      p["aut_W"], p["aut_b"].reshape(1, -1),
      p["com_W"], p["com_b"].reshape(1, -1))
    return out


def _mlp_ln_body(x_ref, W1, b1, g, be, W2, b2, out_ref):
    t = _dot(x_ref[...], W1[...]) + b1[...]
    m = t.mean(-1, keepdims=True)
    v = ((t - m) ** 2).mean(-1, keepdims=True)
    t = (t - m) / jnp.sqrt(v + 1e-5) * g[...] + be[...]
    t = jax.nn.relu(t)
    out_ref[...] = _dot(t, W2[...]) + b2[...]


def _mlp_ln(x, W1, b1, g, be, W2, b2):
    din, dh = W1.shape
    dout = W2.shape[1]
    row = lambda i: (i, 0)
    full = lambda i: (0, 0)
    return pl.pallas_call(
        _mlp_ln_body,
        grid=(B // MB,),
        in_specs=[pl.BlockSpec((MB, din), row),
                  pl.BlockSpec((din, dh), full),
                  pl.BlockSpec((1, dh), full),
                  pl.BlockSpec((1, dh), full),
                  pl.BlockSpec((1, dh), full),
                  pl.BlockSpec((dh, dout), full),
                  pl.BlockSpec((1, dout), full)],
        out_specs=pl.BlockSpec((MB, dout), row),
        out_shape=jax.ShapeDtypeStruct((B, dout), F32),
    )(x, W1, b1.reshape(1, -1), g.reshape(1, -1), be.reshape(1, -1),
      W2, b2.reshape(1, -1))


def _router_stats(gates):
    importance = gates.sum(0, keepdims=True)
    load = (gates > 0.0).astype(F32).sum(0, keepdims=True)

    def cv_sq(v):
        m = v.mean()
        var = ((v - m) ** 2).mean()
        return var / (m * m + 1e-10)

    return (cv_sq(importance) + cv_sq(load)) * 0.01


def _top2_gates(logits):
    lane = jax.lax.broadcasted_iota(jnp.int32, logits.shape, 1)
    i1 = jnp.argmax(logits, axis=1, keepdims=True)
    oh1 = (lane == i1)
    m1 = jnp.max(logits, axis=1, keepdims=True)
    masked = jnp.where(oh1, -jnp.inf, logits)
    i2 = jnp.argmax(masked, axis=1, keepdims=True)
    oh2 = (lane == i2)
    m2 = jnp.max(masked, axis=1, keepdims=True)
    e2 = jnp.exp(m2 - m1)
    den = 1.0 + e2
    g1 = 1.0 / den
    g2 = e2 / den
    return jnp.where(oh1, g1, 0.0) + jnp.where(oh2, g2, 0.0)


def _reduce_body(x_ref, rdW, rdb, fea_ref):
    fea_ref[...] = _dot(x_ref[...], rdW[...]) + rdb[...]


def _reduce(x7, p):
    row = lambda i: (i, 0)
    full = lambda i: (0, 0)
    return pl.pallas_call(
        _reduce_body,
        grid=(B // MB,),
        in_specs=[pl.BlockSpec((MB, 7 * CLS), row),
                  pl.BlockSpec((7 * CLS, MIN), full),
                  pl.BlockSpec((1, MIN), full)],
        out_specs=pl.BlockSpec((MB, MIN), row),
        out_shape=jax.ShapeDtypeStruct((B, MIN), F32),
    )(x7, p["rd_W"], p["rd_b"].reshape(1, -1))


def _router_body(fea_ref, wgc, wgr, gc_ref, gr_ref, lc_ref, lr_ref):
    fea = fea_ref[...]
    gc = _top2_gates(_dot(fea, wgc[...]))
    gr = _top2_gates(_dot(fea, wgr[...]))
    gc_ref[...] = gc
    gr_ref[...] = gr
    lc_ref[0, 0] = _router_stats(gc)
    lr_ref[0, 0] = _router_stats(gr)


def _router(fea, p):
    full = lambda: (0, 0)
    return pl.pallas_call(
        _router_body,
        in_specs=[pl.BlockSpec((B, MIN), full),
                  pl.BlockSpec((MIN, E), full),
                  pl.BlockSpec((MIN, E), full)],
        out_specs=[pl.BlockSpec((B, E), full),
                   pl.BlockSpec((B, E), full),
                   pl.BlockSpec(memory_space=pltpu.SMEM),
                   pl.BlockSpec(memory_space=pltpu.SMEM)],
        out_shape=[jax.ShapeDtypeStruct((B, E), F32),
                   jax.ShapeDtypeStruct((B, E), F32),
                   jax.ShapeDtypeStruct((1, 1), F32),
                   jax.ShapeDtypeStruct((1, 1), F32)],
    )(fea, p["wg_cls"], p["wg_reg"])


def _moe_body(fea_ref, g_ref, W1, b1, W2, b2, out_ref):
    e = pl.program_id(0)
    x = fea_ref[...]
    lane = jax.lax.broadcasted_iota(jnp.int32, (B, E), 1)
    sel = (lane == e).astype(F32)

    h = jax.nn.relu(_dot(x, W1[0]) + b1[0])
    y = _dot(h, W2[0]) + b2[0]
    col = (g_ref[...] * sel).sum(1, keepdims=True)
    y = y.astype(jnp.bfloat16).astype(F32)
    col = col.astype(jnp.bfloat16).astype(F32)

    @pl.when(e == 0)
    def _():
        out_ref[...] = col * y

    @pl.when(e != 0)
    def _():
        out_ref[...] += col * y


def _moe_branch_pallas(fea, g, W1, b1, W2, b2):
    full2 = lambda e: (0, 0)
    we3 = lambda e: (e, 0, 0)
    return pl.pallas_call(
        _moe_body,
        grid=(E,),
        in_specs=[pl.BlockSpec((B, MIN), full2),
                  pl.BlockSpec((B, E), full2),
                  pl.BlockSpec((1, MIN, MH), we3), pl.BlockSpec((1, 1, MH), we3),
                  pl.BlockSpec((1, MH, MO), we3), pl.BlockSpec((1, 1, MO), we3)],
        out_specs=pl.BlockSpec((B, MO), full2),
        out_shape=jax.ShapeDtypeStruct((B, MO), F32),
    )(fea, g, W1, b1.reshape(E, 1, MH), W2, b2.reshape(E, 1, MO))


def _head_body(x_ref, W1, b1, g, be, W2, b2, out_ref):
    t = _dot(x_ref[...], W1[...]) + b1[...]
    m = t.mean(-1, keepdims=True)
    v = ((t - m) ** 2).mean(-1, keepdims=True)
    t = jax.nn.relu((t - m) / jnp.sqrt(v + 1e-5) * g[...] + be[...])
    out_ref[...] = _dot(t, W2[...]) + b2[...]


def _head(x, W1, b1, g, be, W2, b2):
    din, dh = W1.shape
    dout = W2.shape[1]
    return pl.pallas_call(
        _head_body,
        in_specs=[pl.BlockSpec((B, din), lambda: (0, 0)),
                  pl.BlockSpec((din, dh), lambda: (0, 0)),
                  pl.BlockSpec((1, dh), lambda: (0, 0)),
                  pl.BlockSpec((1, dh), lambda: (0, 0)),
                  pl.BlockSpec((1, dh), lambda: (0, 0)),
                  pl.BlockSpec((dh, dout), lambda: (0, 0)),
                  pl.BlockSpec((1, dout), lambda: (0, 0))],
        out_specs=pl.BlockSpec((B, dout), lambda: (0, 0)),
        out_shape=jax.ShapeDtypeStruct((B, dout), F32),
    )(x, W1, b1.reshape(1, -1), g.reshape(1, -1), be.reshape(1, -1),
      W2, b2.reshape(1, -1))


def _gru_body(x_ref, wihT, whhT, bih, bhh, fcW, fcb,
              rgW1, rgb1, rgg, rgbe, rgW2, rgb2, out_ref, gout_ref, X, GI):
    X[...] = x_ref[...]
    for l in range(3):
        GI[...] = _dot(X[...], wihT[l]) + bih[l][None, :]
        bhh_l = bhh[l][None, :]
        whhT_l = whhT[l]

        def step(t, h):
            gh = _dot(h, whhT_l) + bhh_l
            gi = GI[pl.ds(t, 1), :]
            r = jax.nn.sigmoid(gi[:, :CLS] + gh[:, :CLS])
            z = jax.nn.sigmoid(gi[:, CLS:2 * CLS] + gh[:, CLS:2 * CLS])
            n = jnp.tanh(gi[:, 2 * CLS:] + r * gh[:, 2 * CLS:])
            hnew = (1.0 - z) * n + z * h
            X[pl.ds(t, 1), :] = hnew
            return hnew

        jax.lax.fori_loop(0, B, step, jnp.zeros((1, CLS), F32))

    g = _dot(X[...], fcW[...]) + fcb[...]
    gout_ref[...] = g
    t = _dot(g, rgW1[...]) + rgb1[...]
    m = t.mean(-1, keepdims=True)
    v = ((t - m) ** 2).mean(-1, keepdims=True)
    t = jax.nn.relu((t - m) / jnp.sqrt(v + 1e-5) * rgg[...] + rgbe[...])
    out_ref[...] = _dot(t, rgW2[...]) + rgb2[...]


def _gru_reg(reg_fea, p, rgW2p, rgb2p):
    wihT = jnp.transpose(p["gru_Wih"], (0, 2, 1))
    whhT = jnp.transpose(p["gru_Whh"], (0, 2, 1))
    f = lambda: (0, 0)
    f3 = lambda: (0, 0, 0)
    out, _ = pl.pallas_call(
        _gru_body,
        in_specs=[pl.BlockSpec((B, CLS), f),
                  pl.BlockSpec((3, CLS, 3 * CLS), f3),
                  pl.BlockSpec((3, CLS, 3 * CLS), f3),
                  pl.BlockSpec((3, 3 * CLS), f),
                  pl.BlockSpec((3, 3 * CLS), f),
                  pl.BlockSpec((CLS, CLS), f),
                  pl.BlockSpec((1, CLS), f),
                  pl.BlockSpec((CLS, CLS // 2), f),
                  pl.BlockSpec((1, CLS // 2), f),
                  pl.BlockSpec((1, CLS // 2), f),
                  pl.BlockSpec((1, CLS // 2), f),
                  pl.BlockSpec((CLS // 2, 128), f),
                  pl.BlockSpec((1, 128), f)],
        out_specs=[pl.BlockSpec((B, 128), f),
                   pl.BlockSpec((B, CLS), f)],
        out_shape=[jax.ShapeDtypeStruct((B, 128), F32),
                   jax.ShapeDtypeStruct((B, CLS), F32)],
        scratch_shapes=[pltpu.VMEM((B, CLS), F32),
                        pltpu.VMEM((B, 3 * CLS), F32)],
    )(reg_fea, wihT, whhT, p["gru_bih"], p["gru_bhh"],
      p["fc_W"], p["fc_b"].reshape(1, -1),
      p["rg_W1"], p["rg_b1"].reshape(1, -1),
      p["rg_g"].reshape(1, -1), p["rg_be"].reshape(1, -1),
      rgW2p, rgb2p)
    return out


def kernel(video_feas, asr_feas, topics_fea, title_feas, author_feas,
           comment_feas, params):
    p = params
    video2 = video_feas.reshape(B * T, FD)
    com2 = comment_feas[:, :DN, :].reshape(B * DN, FD)

    f1, f2, tpf = _encoders(video2, com2, asr_feas, topics_fea,
                            title_feas, author_feas, p)

    v_p = _mlp_ln(f1, p["vp_W1"], p["vp_b1"], p["vp_g"], p["vp_be"],
                  p["vp_W2"], p["vp_b2"])
    v_c = _mlp_ln(f2, p["vc_W1"], p["vc_b1"], p["vc_g"], p["vc_be"],
                  p["vc_W2"], p["vc_b2"])

    x7 = jnp.concatenate([tpf, v_p, v_c], axis=1)
    fea = _reduce(x7, p)
    gc, gr, lc, lr = _router(fea, p)

    cls_fea = _moe_branch_pallas(fea, gc, p["ec_W1"], p["ec_b1"],
                                 p["ec_W2"], p["ec_b2"])
    reg_fea = _moe_branch_pallas(fea, gr, p["er_W1"], p["er_b1"],
                                 p["er_W2"], p["er_b2"])

    clW2p = jnp.pad(p["cl_W2"], ((0, 0), (0, 126)))
    clb2p = jnp.pad(p["cl_b2"], (0, 126)).reshape(1, -1)
    cls_pad = _head(cls_fea, p["cl_W1"], p["cl_b1"], p["cl_g"], p["cl_be"],
                    clW2p, clb2p.reshape(-1))
    cls_output = cls_pad[:, :2]

    rgW2p = jnp.pad(p["rg_W2"], ((0, 0), (0, 127)))
    rgb2p = jnp.pad(p["rg_b2"], (0, 127)).reshape(1, -1)
    reg_pad = _gru_reg(reg_fea, p, rgW2p, rgb2p)
    reg_output = reg_pad[:, :1]

    return (cls_output, reg_output, lc.reshape(()), lr.reshape(()))
